# baseline (speedup 1.0000x reference)
import functools

import jax
import jax.numpy as jnp
from jax import lax
from jax.experimental import pallas as pl
from jax.experimental.pallas import tpu as pltpu
from jax.experimental.pallas import tpu_sc as plsc

NCORES = 2
NSUB = 16
CH = 64
BLK = 64
FW = 144
MAXC = 161


def _tc_feat_body(x_ref, w_ref, wa_ref, bh_ref, ba_ref, ftab_ref, a_ref):
    f = jnp.dot(x_ref[...], w_ref[...], preferred_element_type=jnp.float32)
    f = f + bh_ref[...]
    ac = jnp.dot(f, wa_ref[...], preferred_element_type=jnp.float32)
    ac = ac + ba_ref[...]
    blk = f.shape[0]
    ones = jnp.ones((blk, 1), jnp.float32)
    zpad = jnp.zeros((blk, FW - 130), jnp.float32)
    ftab_ref[...] = jnp.concatenate([f, ones, ac[:, 1:2], zpad], axis=1)
    a_ref[...] = ac[:, 0:2]


def _tc_norm_body(acc_ref, out_ref):
    blk = acc_ref[...]
    s = blk[:, 128:129]
    out_ref[...] = jnp.where(s > 0.0, blk[:, :128] / s, 0.0)


def _make_sc_kernel(accr, ntab, half):
    mesh = plsc.VectorSubcoreMesh(
        core_axis_name="c", subcore_axis_name="s",
        num_cores=NCORES, num_subcores=NSUB,
    )
    rows_per_sub = accr // NSUB

    @functools.partial(
        pl.kernel,
        out_type=jax.ShapeDtypeStruct(
            (NCORES, NSUB, rows_per_sub, FW), jnp.float32),
        mesh=mesh,
        compiler_params=pltpu.CompilerParams(
            needs_layout_passes=False, use_tc_tiling_on_sc=False),
        scratch_types=[
            pltpu.VMEM((ntab,), jnp.float32),
            pltpu.VMEM((32,), jnp.int32),
            pltpu.VMEM((MAXC, CH), jnp.int32),
            pltpu.VMEM((MAXC, CH), jnp.int32),
            pltpu.VMEM((CH, FW), jnp.float32),
            pltpu.VMEM((CH, FW), jnp.float32),
            pltpu.VMEM((CH, FW), jnp.float32),
            pltpu.VMEM((CH, FW), jnp.float32),
            pltpu.SemaphoreType.DMA,
            pltpu.SemaphoreType.DMA,
            pltpu.SemaphoreType.DMA,
            pltpu.SemaphoreType.DMA,
            pltpu.VMEM_SHARED((accr, FW), jnp.float32),
        ],
    )
    def sc_kernel(ftab_hbm, a_hbm, src_hbm, dst_hbm, ncw_hbm,
                  acc_hbm, a_v, ncw_v, src_v, dst_v,
                  inA, inB, outA, outB,
                  gsA, gsB, ssA, ssB, acc_s):
        cid = lax.axis_index("c")
        sid = lax.axis_index("s")

        def g_desc(ci, buf, sem):
            return pltpu.make_async_copy(ftab_hbm.at[dst_v.at[ci]], buf, sem)

        def s_desc(ci, buf, sem):
            return pltpu.make_async_copy(buf, acc_s.at[src_v.at[ci]], sem)

        pltpu.sync_copy(a_hbm, a_v)
        pltpu.sync_copy(src_hbm.at[cid, sid], src_v)
        pltpu.sync_copy(dst_hbm.at[cid, sid], dst_v)
        pltpu.sync_copy(ncw_hbm, ncw_v)

        iota16 = lax.iota(jnp.int32, 16)
        sid16 = jnp.full((16,), sid, jnp.int32)
        w16 = jnp.where(iota16 == sid16,
                        ncw_v[pl.ds(0, 16)] * (1 - cid) +
                        ncw_v[pl.ds(16, 16)] * cid,
                        0)
        nchw = jnp.sum(w16, axis=0)
        pairs = jnp.maximum((nchw + 1) >> 1, 1)

        g_desc(0, inA, gsA).start()
        g_desc(1, inB, gsB).start()

        zero16 = jnp.zeros((16,), jnp.float32)

        def zrow(i, carry):
            for v in range(FW // 16):
                outA[i, pl.ds(v * 16, 16)] = zero16
            return carry

        lax.fori_loop(0, CH, zrow, 0)
        row0 = sid * rows_per_sub
        nfull, rem = divmod(rows_per_sub, CH)
        for k in range(nfull):
            pltpu.sync_copy(outA, acc_s.at[pl.ds(row0 + k * CH, CH)])
        if rem:
            pltpu.sync_copy(outA.at[pl.ds(0, rem)],
                            acc_s.at[pl.ds(row0 + nfull * CH, rem)])

        plsc.subcore_barrier()

        abase = jnp.full((16,), cid * half, jnp.int32)

        def scale(ci, inb, outb):
            for j in range(CH // 16):
                base = j * 16
                s16 = src_v[ci, pl.ds(base, 16)]
                a16 = plsc.load_gather(a_v, [s16 + abase])
                for l in range(16):
                    crow = inb[base + l, pl.ds(128, 16)]
                    xs = jnp.full((16,), a16[l] + crow[1], jnp.float32)
                    ps = jnp.exp(jnp.maximum(xs, 0.2 * xs))
                    for v in range(FW // 16):
                        sl = pl.ds(v * 16, 16)
                        outb[base + l, sl] = inb[base + l, sl] * ps

        bufs = [(inA, outA, gsA, ssA), (inB, outB, gsB, ssB)]

        def pair_body(c2, carry):
            for b, (inb, outb, gs, ss) in enumerate(bufs):
                ci = 2 * c2 + b
                g_desc(ci, inb, gs).wait()

                @pl.when(c2 > 0)
                def _():
                    s_desc(ci - 2, outb, ss).wait()

                scale(ci, inb, outb)
                g_desc(ci + 2, inb, gs).start()
                s_desc(ci, outb, ss).start(add=True)
            return carry

        lax.fori_loop(0, pairs, pair_body, 0)

        last = 2 * pairs
        s_desc(last - 2, outA, ssA).wait()
        s_desc(last - 1, outB, ssB).wait()
        g_desc(last, inA, gsA).wait()
        g_desc(last + 1, inB, gsB).wait()

        plsc.subcore_barrier()

        pltpu.sync_copy(acc_s.at[pl.ds(row0, rows_per_sub)],
                        acc_hbm.at[cid, sid])

    return sc_kernel


def kernel(node_features, edges, W_hidden, b_hidden, W_att, b_att):
    n, d = node_features.shape
    h = W_hidden.shape[1]
    e = edges.shape[0]

    npad = ((n + 1 + BLK - 1) // BLK) * BLK
    half = npad // 2
    accr = ((half + 1 + NSUB - 1) // NSUB) * NSUB
    ntab = npad + 16
    slots = (MAXC - 4) * CH

    xp = jnp.pad(node_features, ((0, npad - n), (0, 0)))
    wa = W_att.reshape(2, h).T
    bh2 = b_hidden.reshape(1, h)
    ba2 = jnp.concatenate([b_att, jnp.zeros((1,), jnp.float32)]).reshape(1, 2)

    src = edges[:, 0]
    dst = edges[:, 1]
    bnd = jnp.searchsorted(src, half).astype(jnp.int32)
    cnt = jnp.stack([bnd, e - bnd])
    per_w = (cnt + NSUB - 1) // NSUB
    c_idx = jnp.arange(NCORES, dtype=jnp.int32)[:, None, None]
    w_idx = jnp.arange(NSUB, dtype=jnp.int32)[None, :, None]
    j_idx = jnp.arange(slots, dtype=jnp.int32)[None, None, :]
    off = jnp.stack([jnp.zeros((), jnp.int32), bnd])[:, None, None]
    vw = jnp.clip(cnt[:, None, None] - w_idx * per_w[:, None, None],
                  0, per_w[:, None, None])
    eidx = off + w_idx * per_w[:, None, None] + j_idx
    valid = j_idx < vw
    eidx_c = jnp.minimum(eidx, e - 1)
    srcl = jnp.where(valid, src[eidx_c] - c_idx * half, half).astype(jnp.int32)
    dsta = jnp.where(valid, dst[eidx_c], n).astype(jnp.int32)
    padc = jnp.full((NCORES, NSUB, 4 * CH), n, jnp.int32)
    srcl = jnp.concatenate(
        [srcl, jnp.full((NCORES, NSUB, 4 * CH), half, jnp.int32)], axis=2)
    dsta = jnp.concatenate([dsta, padc], axis=2)
    src_p = srcl.reshape(NCORES, NSUB, MAXC, CH)
    dst_p = dsta.reshape(NCORES, NSUB, MAXC, CH)
    ncw = ((vw[:, :, 0] + CH - 1) // CH).astype(jnp.int32).reshape(-1)

    ftab, ac = pl.pallas_call(
        _tc_feat_body,
        grid=(npad // BLK,),
        in_specs=[
            pl.BlockSpec((BLK, d), lambda i: (i, 0)),
            pl.BlockSpec((d, h), lambda i: (0, 0)),
            pl.BlockSpec((h, 2), lambda i: (0, 0)),
            pl.BlockSpec((1, h), lambda i: (0, 0)),
            pl.BlockSpec((1, 2), lambda i: (0, 0)),
        ],
        out_specs=[
            pl.BlockSpec((BLK, FW), lambda i: (i, 0)),
            pl.BlockSpec((BLK, 2), lambda i: (i, 0)),
        ],
        out_shape=[
            jax.ShapeDtypeStruct((npad, FW), jnp.float32),
            jax.ShapeDtypeStruct((npad, 2), jnp.float32),
        ],
    )(xp, W_hidden, wa, bh2, ba2)

    a_tab = jnp.pad(ac[:, 0], (0, ntab - npad))

    acc = _make_sc_kernel(accr, ntab, half)(
        ftab, a_tab, src_p, dst_p, ncw)
    acc = acc.reshape(NCORES, accr, FW)
    acc_g = jnp.concatenate([acc[0, :half], acc[1, :half]], axis=0)

    out = pl.pallas_call(
        _tc_norm_body,
        grid=(npad // BLK,),
        in_specs=[pl.BlockSpec((BLK, FW), lambda i: (i, 0))],
        out_specs=pl.BlockSpec((BLK, h), lambda i: (i, 0)),
        out_shape=jax.ShapeDtypeStruct((npad, h), jnp.float32),
    )(acc_g)

    return out[:n]


# speedup vs baseline: 1.0003x; 1.0003x over previous
import functools

import jax
import jax.numpy as jnp
from jax import lax
from jax.experimental import pallas as pl
from jax.experimental.pallas import tpu as pltpu
from jax.experimental.pallas import tpu_sc as plsc

NCORES = 2
NSUB = 16
CH = 64
BLK = 64
MAXC = 161


def _tc_feat_body(x_ref, w_ref, wa_ref, bh_ref, ba_ref, feat_ref, ac_ref):
    f = jnp.dot(x_ref[...], w_ref[...], preferred_element_type=jnp.float32)
    f = f + bh_ref[...]
    feat_ref[...] = f
    ac_ref[...] = (
        jnp.dot(f, wa_ref[...], preferred_element_type=jnp.float32) + ba_ref[...]
    )


def _tc_norm_body(acc_ref, s_ref, out_ref):
    s = s_ref[...]
    out_ref[...] = jnp.where(s > 0.0, acc_ref[...] / s, 0.0)


def _make_sc_kernel(accr, nsum, ntab, half, hdim):
    mesh = plsc.VectorSubcoreMesh(
        core_axis_name="c", subcore_axis_name="s",
        num_cores=NCORES, num_subcores=NSUB,
    )
    rows_per_sub = accr // NSUB
    srows_per_sub = nsum // NSUB

    @functools.partial(
        pl.kernel,
        out_type=[
            jax.ShapeDtypeStruct(
                (NCORES, NSUB, rows_per_sub, hdim), jnp.float32),
            jax.ShapeDtypeStruct((NCORES * nsum,), jnp.float32),
        ],
        mesh=mesh,
        compiler_params=pltpu.CompilerParams(
            needs_layout_passes=False, use_tc_tiling_on_sc=False),
        scratch_types=[
            pltpu.VMEM((ntab,), jnp.float32),
            pltpu.VMEM((ntab,), jnp.float32),
            pltpu.VMEM((32,), jnp.int32),
            pltpu.VMEM((MAXC, CH), jnp.int32),
            pltpu.VMEM((MAXC, CH), jnp.int32),
            pltpu.VMEM((CH, hdim), jnp.float32),
            pltpu.VMEM((CH, hdim), jnp.float32),
            pltpu.VMEM((CH, hdim), jnp.float32),
            pltpu.VMEM((CH, hdim), jnp.float32),
            pltpu.VMEM((CH,), jnp.float32),
            pltpu.VMEM((CH,), jnp.float32),
            pltpu.VMEM((640,), jnp.float32),
            pltpu.SemaphoreType.DMA,
            pltpu.SemaphoreType.DMA,
            pltpu.SemaphoreType.DMA,
            pltpu.SemaphoreType.DMA,
            pltpu.SemaphoreType.DMA,
            pltpu.SemaphoreType.DMA,
            pltpu.VMEM_SHARED((accr, hdim), jnp.float32),
            pltpu.VMEM_SHARED((nsum,), jnp.float32),
        ],
    )
    def sc_kernel(feat_hbm, a_hbm, c_hbm, src_hbm, dst_hbm, ncw_hbm,
                  acc_hbm, sums_hbm, a_v, c_v, ncw_v, src_v, dst_v,
                  inA, inB, outA, outB, pA, pB, st_v,
                  gsA, gsB, ssA, ssB, usA, usB, acc_s, sums_s):
        cid = lax.axis_index("c")
        sid = lax.axis_index("s")

        def g_desc(ci, buf, sem):
            return pltpu.make_async_copy(feat_hbm.at[dst_v.at[ci]], buf, sem)

        def s_desc(ci, buf, sem):
            return pltpu.make_async_copy(buf, acc_s.at[src_v.at[ci]], sem)

        def u_desc(ci, pbuf, sem):
            return pltpu.make_async_copy(pbuf, sums_s.at[src_v.at[ci]], sem)

        pltpu.sync_copy(a_hbm, a_v)
        pltpu.sync_copy(c_hbm, c_v)
        pltpu.sync_copy(src_hbm.at[cid, sid], src_v)
        pltpu.sync_copy(dst_hbm.at[cid, sid], dst_v)
        pltpu.sync_copy(ncw_hbm, ncw_v)

        iota16 = lax.iota(jnp.int32, 16)
        sid16 = jnp.full((16,), sid, jnp.int32)
        w16 = jnp.where(iota16 == sid16,
                        ncw_v[pl.ds(0, 16)] * (1 - cid) +
                        ncw_v[pl.ds(16, 16)] * cid,
                        0)
        nchw = jnp.sum(w16, axis=0)
        pairs = jnp.maximum((nchw + 1) >> 1, 1)

        g_desc(0, inA, gsA).start()
        g_desc(1, inB, gsB).start()

        zero16 = jnp.zeros((16,), jnp.float32)

        def zrow(i, carry):
            for v in range(hdim // 16):
                outA[i, pl.ds(v * 16, 16)] = zero16
            return carry

        lax.fori_loop(0, CH, zrow, 0)

        def zst(i, carry):
            st_v[pl.ds(i * 16, 16)] = zero16
            return carry

        lax.fori_loop(0, 640 // 16, zst, 0)

        row0 = sid * rows_per_sub
        nfull, rem = divmod(rows_per_sub, CH)
        for k in range(nfull):
            pltpu.sync_copy(outA, acc_s.at[pl.ds(row0 + k * CH, CH)])
        if rem:
            pltpu.sync_copy(outA.at[pl.ds(0, rem)],
                            acc_s.at[pl.ds(row0 + nfull * CH, rem)])
        srow0 = sid * srows_per_sub
        pltpu.sync_copy(st_v.at[pl.ds(0, srows_per_sub)],
                        sums_s.at[pl.ds(srow0, srows_per_sub)])

        plsc.subcore_barrier()

        abase = jnp.full((16,), cid * half, jnp.int32)

        def scale(ci, inb, outb, pbuf):
            for j in range(CH // 16):
                base = j * 16
                s16 = src_v[ci, pl.ds(base, 16)]
                d16 = dst_v[ci, pl.ds(base, 16)]
                av = plsc.load_gather(a_v, [s16 + abase])
                cv = plsc.load_gather(c_v, [d16])
                x = av + cv
                p16 = jnp.exp(jnp.maximum(x, 0.2 * x))
                pbuf[pl.ds(base, 16)] = p16
                for l in range(16):
                    ps = jnp.full((16,), p16[l], jnp.float32)
                    for v in range(hdim // 16):
                        sl = pl.ds(v * 16, 16)
                        outb[base + l, sl] = inb[base + l, sl] * ps

        bufs = [(inA, outA, pA, gsA, ssA, usA),
                (inB, outB, pB, gsB, ssB, usB)]

        def pair_body(c2, carry):
            for b, (inb, outb, pbuf, gs, ss, us) in enumerate(bufs):
                ci = 2 * c2 + b
                g_desc(ci, inb, gs).wait()

                @pl.when(c2 > 0)
                def _():
                    s_desc(ci - 2, outb, ss).wait()
                    u_desc(ci - 2, pbuf, us).wait()

                scale(ci, inb, outb, pbuf)
                g_desc(ci + 2, inb, gs).start()
                s_desc(ci, outb, ss).start(add=True)
                u_desc(ci, pbuf, us).start(add=True)
            return carry

        lax.fori_loop(0, pairs, pair_body, 0)

        last = 2 * pairs
        s_desc(last - 2, outA, ssA).wait()
        u_desc(last - 2, pA, usA).wait()
        s_desc(last - 1, outB, ssB).wait()
        u_desc(last - 1, pB, usB).wait()
        g_desc(last, inA, gsA).wait()
        g_desc(last + 1, inB, gsB).wait()

        plsc.subcore_barrier()

        pltpu.sync_copy(acc_s.at[pl.ds(row0, rows_per_sub)],
                        acc_hbm.at[cid, sid])
        pltpu.sync_copy(sums_s.at[pl.ds(srow0, srows_per_sub)],
                        st_v.at[pl.ds(0, srows_per_sub)])
        pltpu.sync_copy(st_v.at[pl.ds(0, srows_per_sub)],
                        sums_hbm.at[pl.ds(cid * nsum + srow0, srows_per_sub)])

    return sc_kernel


def kernel(node_features, edges, W_hidden, b_hidden, W_att, b_att):
    n, d = node_features.shape
    h = W_hidden.shape[1]
    e = edges.shape[0]

    npad = ((n + 1 + BLK - 1) // BLK) * BLK
    half = npad // 2
    accr = ((half + 1 + NSUB - 1) // NSUB) * NSUB
    nsum = ((half + 1 + 127) // 128) * 128
    ntab = npad + 16
    slots = (MAXC - 4) * CH

    xp = jnp.pad(node_features, ((0, npad - n), (0, 0)))
    wa = W_att.reshape(2, h).T
    bh2 = b_hidden.reshape(1, h)
    ba2 = jnp.concatenate([b_att, jnp.zeros((1,), jnp.float32)]).reshape(1, 2)

    src = edges[:, 0]
    dst = edges[:, 1]
    bnd = jnp.searchsorted(src, half).astype(jnp.int32)
    cnt = jnp.stack([bnd, e - bnd])
    per_w = (cnt + NSUB - 1) // NSUB
    c_idx = jnp.arange(NCORES, dtype=jnp.int32)[:, None, None]
    w_idx = jnp.arange(NSUB, dtype=jnp.int32)[None, :, None]
    j_idx = jnp.arange(slots, dtype=jnp.int32)[None, None, :]
    off = jnp.stack([jnp.zeros((), jnp.int32), bnd])[:, None, None]
    vw = jnp.clip(cnt[:, None, None] - w_idx * per_w[:, None, None],
                  0, per_w[:, None, None])
    eidx = off + w_idx * per_w[:, None, None] + j_idx
    valid = j_idx < vw
    eidx_c = jnp.minimum(eidx, e - 1)
    srcl = jnp.where(valid, src[eidx_c] - c_idx * half, half).astype(jnp.int32)
    dsta = jnp.where(valid, dst[eidx_c], n).astype(jnp.int32)
    srcl = jnp.concatenate(
        [srcl, jnp.full((NCORES, NSUB, 4 * CH), half, jnp.int32)], axis=2)
    dsta = jnp.concatenate(
        [dsta, jnp.full((NCORES, NSUB, 4 * CH), n, jnp.int32)], axis=2)
    src_p = srcl.reshape(NCORES, NSUB, MAXC, CH)
    dst_p = dsta.reshape(NCORES, NSUB, MAXC, CH)
    ncw = ((vw[:, :, 0] + CH - 1) // CH).astype(jnp.int32).reshape(-1)

    feat, ac = pl.pallas_call(
        _tc_feat_body,
        grid=(npad // BLK,),
        in_specs=[
            pl.BlockSpec((BLK, d), lambda i: (i, 0)),
            pl.BlockSpec((d, h), lambda i: (0, 0)),
            pl.BlockSpec((h, 2), lambda i: (0, 0)),
            pl.BlockSpec((1, h), lambda i: (0, 0)),
            pl.BlockSpec((1, 2), lambda i: (0, 0)),
        ],
        out_specs=[
            pl.BlockSpec((BLK, h), lambda i: (i, 0)),
            pl.BlockSpec((BLK, 2), lambda i: (i, 0)),
        ],
        out_shape=[
            jax.ShapeDtypeStruct((npad, h), jnp.float32),
            jax.ShapeDtypeStruct((npad, 2), jnp.float32),
        ],
    )(xp, W_hidden, wa, bh2, ba2)

    a_tab = jnp.pad(ac[:, 0], (0, ntab - npad))
    c_tab = jnp.pad(ac[:, 1], (0, ntab - npad))

    acc, sums = _make_sc_kernel(accr, nsum, ntab, half, h)(
        feat, a_tab, c_tab, src_p, dst_p, ncw)
    acc = acc.reshape(NCORES, accr, h)
    acc_g = jnp.concatenate([acc[0, :half], acc[1, :half]], axis=0)
    s2 = sums.reshape(NCORES, nsum)
    s_g = jnp.concatenate([s2[0, :half], s2[1, :half]], axis=0)[:, None]

    out = pl.pallas_call(
        _tc_norm_body,
        grid=(npad // BLK,),
        in_specs=[
            pl.BlockSpec((BLK, h), lambda i: (i, 0)),
            pl.BlockSpec((BLK, 1), lambda i: (i, 0)),
        ],
        out_specs=pl.BlockSpec((BLK, h), lambda i: (i, 0)),
        out_shape=jax.ShapeDtypeStruct((npad, h), jnp.float32),
    )(acc_g, s_g)

    return out[:n]


# speedup vs baseline: 29.3000x; 29.2915x over previous
import functools

import jax
import jax.numpy as jnp
from jax import lax
from jax.experimental import pallas as pl
from jax.experimental.pallas import tpu as pltpu
from jax.experimental.pallas import tpu_sc as plsc

NCORES = 2
NSUB = 16
CH = 64
BLK = 64
MAXC = 161


def _tc_feat_body(x_ref, w_ref, wa_ref, bh_ref, ba_ref, feat_ref, ac_ref):
    f = jnp.dot(x_ref[...], w_ref[...], preferred_element_type=jnp.float32)
    f = f + bh_ref[...]
    feat_ref[...] = f
    ac_ref[...] = (
        jnp.dot(f, wa_ref[...], preferred_element_type=jnp.float32) + ba_ref[...]
    )


def _tc_norm_body(acc_ref, s_ref, out_ref):
    s = s_ref[...]
    out_ref[...] = jnp.where(s > 0.0, acc_ref[...] / s, 0.0)


def _make_sc_kernel(accr, nsum, ntab, half, hdim):
    mesh = plsc.VectorSubcoreMesh(
        core_axis_name="c", subcore_axis_name="s",
        num_cores=NCORES, num_subcores=NSUB,
    )
    rows_per_sub = accr // NSUB
    srows_per_sub = nsum // NSUB

    @functools.partial(
        pl.kernel,
        out_type=[
            jax.ShapeDtypeStruct(
                (NCORES, NSUB, rows_per_sub, hdim), jnp.float32),
            jax.ShapeDtypeStruct((NCORES * nsum,), jnp.float32),
        ],
        mesh=mesh,
        compiler_params=pltpu.CompilerParams(
            needs_layout_passes=False, use_tc_tiling_on_sc=False),
        scratch_types=[
            pltpu.VMEM((ntab,), jnp.float32),
            pltpu.VMEM((ntab,), jnp.float32),
            pltpu.VMEM((32,), jnp.int32),
            pltpu.VMEM((MAXC, CH), jnp.int32),
            pltpu.VMEM((MAXC, CH), jnp.int32),
            pltpu.VMEM((CH, hdim), jnp.float32),
            pltpu.VMEM((CH, hdim), jnp.float32),
            pltpu.VMEM((CH, hdim), jnp.float32),
            pltpu.VMEM((CH, hdim), jnp.float32),
            pltpu.VMEM((CH,), jnp.float32),
            pltpu.VMEM((CH,), jnp.float32),
            pltpu.VMEM((640,), jnp.float32),
            pltpu.SemaphoreType.DMA,
            pltpu.SemaphoreType.DMA,
            pltpu.SemaphoreType.DMA,
            pltpu.SemaphoreType.DMA,
            pltpu.SemaphoreType.DMA,
            pltpu.SemaphoreType.DMA,
            pltpu.VMEM_SHARED((accr, hdim), jnp.float32),
            pltpu.VMEM_SHARED((nsum,), jnp.float32),
        ],
    )
    def sc_kernel(feat_hbm, a_hbm, c_hbm, src_hbm, dst_hbm, ncw_hbm,
                  acc_hbm, sums_hbm, a_v, c_v, ncw_v, src_v, dst_v,
                  inA, inB, outA, outB, pA, pB, st_v,
                  gsA, gsB, ssA, ssB, usA, usB, acc_s, sums_s):
        cid = lax.axis_index("c")
        sid = lax.axis_index("s")

        def g_desc(ci, buf, sem):
            return pltpu.make_async_copy(feat_hbm.at[dst_v.at[ci]], buf, sem)

        def s_desc(ci, buf, sem):
            return pltpu.make_async_copy(buf, acc_s.at[src_v.at[ci]], sem)

        def u_desc(ci, pbuf, sem):
            return pltpu.make_async_copy(pbuf, sums_s.at[src_v.at[ci]], sem)

        pltpu.sync_copy(a_hbm, a_v)
        pltpu.sync_copy(c_hbm, c_v)
        pltpu.sync_copy(src_hbm.at[cid, sid], src_v)
        pltpu.sync_copy(dst_hbm.at[cid, sid], dst_v)
        pltpu.sync_copy(ncw_hbm, ncw_v)

        iota16 = lax.iota(jnp.int32, 16)
        sid16 = jnp.full((16,), sid, jnp.int32)
        w16 = jnp.where(iota16 == sid16,
                        ncw_v[pl.ds(0, 16)] * (1 - cid) +
                        ncw_v[pl.ds(16, 16)] * cid,
                        0)
        nchw = jnp.sum(w16, axis=0)
        pairs = jnp.maximum((nchw + 1) >> 1, 1)

        g_desc(0, inA, gsA).start()
        g_desc(1, inB, gsB).start()

        zero16 = jnp.zeros((16,), jnp.float32)

        def zrow(i, carry):
            for v in range(hdim // 16):
                outA[i, pl.ds(v * 16, 16)] = zero16
            return carry

        lax.fori_loop(0, CH, zrow, 0)

        def zst(i, carry):
            st_v[pl.ds(i * 16, 16)] = zero16
            return carry

        lax.fori_loop(0, 640 // 16, zst, 0)

        row0 = sid * rows_per_sub
        nfull, rem = divmod(rows_per_sub, CH)
        for k in range(nfull):
            pltpu.sync_copy(outA, acc_s.at[pl.ds(row0 + k * CH, CH)])
        if rem:
            pltpu.sync_copy(outA.at[pl.ds(0, rem)],
                            acc_s.at[pl.ds(row0 + nfull * CH, rem)])
        srow0 = sid * srows_per_sub
        pltpu.sync_copy(st_v.at[pl.ds(0, srows_per_sub)],
                        sums_s.at[pl.ds(srow0, srows_per_sub)])

        plsc.subcore_barrier()

        abase = jnp.full((16,), cid * half, jnp.int32)

        def scale(ci, inb, outb, pbuf):
            for j in range(CH // 16):
                base = j * 16
                s16 = src_v[ci, pl.ds(base, 16)]
                d16 = dst_v[ci, pl.ds(base, 16)]
                av = plsc.load_gather(a_v, [s16 + abase])
                cv = plsc.load_gather(c_v, [d16])
                x = av + cv
                p16 = jnp.exp(jnp.maximum(x, 0.2 * x))
                pbuf[pl.ds(base, 16)] = p16
                for l in range(16):
                    ps = jnp.full((16,), p16[l], jnp.float32)
                    for v in range(hdim // 16):
                        sl = pl.ds(v * 16, 16)
                        outb[base + l, sl] = inb[base + l, sl] * ps

        bufs = [(inA, outA, pA, gsA, ssA, usA),
                (inB, outB, pB, gsB, ssB, usB)]

        def pair_body(c2, carry):
            for b, (inb, outb, pbuf, gs, ss, us) in enumerate(bufs):
                ci = 2 * c2 + b
                g_desc(ci, inb, gs).wait()

                @pl.when(c2 > 0)
                def _():
                    s_desc(ci - 2, outb, ss).wait()
                    u_desc(ci - 2, pbuf, us).wait()

                scale(ci, inb, outb, pbuf)
                g_desc(ci + 2, inb, gs).start()
                s_desc(ci, outb, ss).start(add=True)
                u_desc(ci, pbuf, us).start(add=True)
            return carry

        lax.fori_loop(0, pairs, pair_body, 0)

        last = 2 * pairs
        s_desc(last - 2, outA, ssA).wait()
        u_desc(last - 2, pA, usA).wait()
        s_desc(last - 1, outB, ssB).wait()
        u_desc(last - 1, pB, usB).wait()
        g_desc(last, inA, gsA).wait()
        g_desc(last + 1, inB, gsB).wait()

        plsc.subcore_barrier()

        pltpu.sync_copy(acc_s.at[pl.ds(row0, rows_per_sub)],
                        acc_hbm.at[cid, sid])
        pltpu.sync_copy(sums_s.at[pl.ds(srow0, srows_per_sub)],
                        st_v.at[pl.ds(0, srows_per_sub)])
        pltpu.sync_copy(st_v.at[pl.ds(0, srows_per_sub)],
                        sums_hbm.at[pl.ds(cid * nsum + srow0, srows_per_sub)])

    return sc_kernel


def kernel(node_features, edges, W_hidden, b_hidden, W_att, b_att):
    n, d = node_features.shape
    h = W_hidden.shape[1]
    e = edges.shape[0]

    npad = ((n + 1 + BLK - 1) // BLK) * BLK
    half = npad // 2
    accr = ((half + 1 + NSUB - 1) // NSUB) * NSUB
    nsum = ((half + 1 + 127) // 128) * 128
    ntab = npad + 16
    slots = (MAXC - 4) * CH

    xp = jnp.pad(node_features, ((0, npad - n), (0, 0)))
    wa = W_att.reshape(2, h).T
    bh2 = b_hidden.reshape(1, h)
    ba2 = jnp.concatenate([b_att, jnp.zeros((1,), jnp.float32)]).reshape(1, 2)

    src = edges[:, 0]
    dst = edges[:, 1]
    bnd = jnp.sum((src < half).astype(jnp.int32))
    cnt = jnp.stack([bnd, e - bnd])
    per_w = (cnt + NSUB - 1) // NSUB
    c_idx = jnp.arange(NCORES, dtype=jnp.int32)[:, None]
    w_idx = jnp.arange(NSUB, dtype=jnp.int32)[None, :]
    j_idx = jnp.arange(slots, dtype=jnp.int32)[None, :]
    off = jnp.stack([jnp.zeros((), jnp.int32), bnd])[:, None]
    vw = jnp.clip(cnt[:, None] - w_idx * per_w[:, None],
                  0, per_w[:, None])
    starts = (off + w_idx * per_w[:, None]).reshape(-1)
    src_x = jnp.concatenate([src, jnp.zeros((slots,), jnp.int32)])
    dst_x = jnp.concatenate([dst, jnp.zeros((slots,), jnp.int32)])

    def _slice(st):
        return (lax.dynamic_slice(src_x, (st,), (slots,)),
                lax.dynamic_slice(dst_x, (st,), (slots,)))

    sl_src, sl_dst = lax.map(_slice, starts)
    sl_src = sl_src.reshape(NCORES, NSUB, slots)
    sl_dst = sl_dst.reshape(NCORES, NSUB, slots)
    valid = j_idx[None] < vw[:, :, None]
    srcl = jnp.where(
        valid,
        sl_src - jnp.arange(NCORES, dtype=jnp.int32)[:, None, None] * half,
        half).astype(jnp.int32)
    dsta = jnp.where(valid, sl_dst, n).astype(jnp.int32)
    srcl = jnp.concatenate(
        [srcl, jnp.full((NCORES, NSUB, 4 * CH), half, jnp.int32)], axis=2)
    dsta = jnp.concatenate(
        [dsta, jnp.full((NCORES, NSUB, 4 * CH), n, jnp.int32)], axis=2)
    src_p = srcl.reshape(NCORES, NSUB, MAXC, CH)
    dst_p = dsta.reshape(NCORES, NSUB, MAXC, CH)
    ncw = ((vw + CH - 1) // CH).astype(jnp.int32).reshape(-1)

    feat, ac = pl.pallas_call(
        _tc_feat_body,
        grid=(npad // BLK,),
        in_specs=[
            pl.BlockSpec((BLK, d), lambda i: (i, 0)),
            pl.BlockSpec((d, h), lambda i: (0, 0)),
            pl.BlockSpec((h, 2), lambda i: (0, 0)),
            pl.BlockSpec((1, h), lambda i: (0, 0)),
            pl.BlockSpec((1, 2), lambda i: (0, 0)),
        ],
        out_specs=[
            pl.BlockSpec((BLK, h), lambda i: (i, 0)),
            pl.BlockSpec((BLK, 2), lambda i: (i, 0)),
        ],
        out_shape=[
            jax.ShapeDtypeStruct((npad, h), jnp.float32),
            jax.ShapeDtypeStruct((npad, 2), jnp.float32),
        ],
    )(xp, W_hidden, wa, bh2, ba2)

    a_tab = jnp.pad(ac[:, 0], (0, ntab - npad))
    c_tab = jnp.pad(ac[:, 1], (0, ntab - npad))

    acc, sums = _make_sc_kernel(accr, nsum, ntab, half, h)(
        feat, a_tab, c_tab, src_p, dst_p, ncw)
    acc = acc.reshape(NCORES, accr, h)
    acc_g = jnp.concatenate([acc[0, :half], acc[1, :half]], axis=0)
    s2 = sums.reshape(NCORES, nsum)
    s_g = jnp.concatenate([s2[0, :half], s2[1, :half]], axis=0)[:, None]

    out = pl.pallas_call(
        _tc_norm_body,
        grid=(npad // BLK,),
        in_specs=[
            pl.BlockSpec((BLK, h), lambda i: (i, 0)),
            pl.BlockSpec((BLK, 1), lambda i: (i, 0)),
        ],
        out_specs=pl.BlockSpec((BLK, h), lambda i: (i, 0)),
        out_shape=jax.ShapeDtypeStruct((npad, h), jnp.float32),
    )(acc_g, s_g)

    return out[:n]


# speedup vs baseline: 32.8883x; 1.1225x over previous
import functools

import jax
import jax.numpy as jnp
from jax import lax
from jax.experimental import pallas as pl
from jax.experimental.pallas import tpu as pltpu
from jax.experimental.pallas import tpu_sc as plsc

NCORES = 2
NSUB = 16
CH = 64
BLK = 64
MAXC = 157


def _tc_feat_body(x_ref, w_ref, wa_ref, bh_ref, ba_ref, feat_ref, ac_ref):
    f = jnp.dot(x_ref[...], w_ref[...], preferred_element_type=jnp.float32)
    f = f + bh_ref[...]
    feat_ref[...] = f
    ac_ref[...] = (
        jnp.dot(f, wa_ref[...], preferred_element_type=jnp.float32) + ba_ref[...]
    )


def _tc_norm_body(acc_ref, s_ref, out_ref):
    s = s_ref[...]
    out_ref[...] = jnp.where(s > 0.0, acc_ref[...] / s, 0.0)


def _make_sc_kernel(accr, ntab, half, hdim):
    mesh = plsc.VectorSubcoreMesh(
        core_axis_name="c", subcore_axis_name="s",
        num_cores=NCORES, num_subcores=NSUB,
    )
    rows_per_sub = accr // NSUB

    @functools.partial(
        pl.kernel,
        out_type=[
            jax.ShapeDtypeStruct(
                (NCORES, NSUB, rows_per_sub, hdim), jnp.float32),
            jax.ShapeDtypeStruct((NCORES, NSUB, accr), jnp.float32),
        ],
        mesh=mesh,
        compiler_params=pltpu.CompilerParams(
            needs_layout_passes=False, use_tc_tiling_on_sc=False),
        scratch_types=[
            pltpu.VMEM((ntab,), jnp.float32),
            pltpu.VMEM((ntab,), jnp.float32),
            pltpu.VMEM((32,), jnp.int32),
            pltpu.VMEM((MAXC, CH), jnp.int32),
            pltpu.VMEM((MAXC, CH), jnp.int32),
            pltpu.VMEM((MAXC, 16), jnp.int32),
            pltpu.VMEM((CH, hdim), jnp.float32),
            pltpu.VMEM((accr,), jnp.float32),
            pltpu.VMEM_SHARED((accr, hdim), jnp.float32),
        ],
    )
    def sc_kernel(feat_hbm, a_hbm, c_hbm, src_hbm, dst_hbm, srcnx_hbm,
                  ncw_hbm, acc_hbm, sums_hbm, a_v, c_v, ncw_v, src_v, dst_v,
                  srcnx_v, msg_v, sums_v, acc_s):
        cid = lax.axis_index("c")
        sid = lax.axis_index("s")

        pltpu.sync_copy(a_hbm, a_v)
        pltpu.sync_copy(c_hbm, c_v)
        pltpu.sync_copy(src_hbm.at[cid, sid], src_v)
        pltpu.sync_copy(dst_hbm.at[cid, sid], dst_v)
        pltpu.sync_copy(srcnx_hbm.at[cid, sid], srcnx_v)
        pltpu.sync_copy(ncw_hbm, ncw_v)

        iota16 = lax.iota(jnp.int32, 16)
        sid16 = jnp.full((16,), sid, jnp.int32)
        w16 = jnp.where(iota16 == sid16,
                        ncw_v[pl.ds(0, 16)] * (1 - cid) +
                        ncw_v[pl.ds(16, 16)] * cid,
                        0)
        nchw = jnp.sum(w16, axis=0)

        zero16 = jnp.zeros((16,), jnp.float32)

        def zrow(i, carry):
            for v in range(hdim // 16):
                msg_v[i, pl.ds(v * 16, 16)] = zero16
            return carry

        lax.fori_loop(0, CH, zrow, 0)

        def zsum(i, carry):
            sums_v[pl.ds(i * 16, 16)] = zero16
            return carry

        lax.fori_loop(0, accr // 16, zsum, 0)

        row0 = sid * rows_per_sub
        nfull, rem = divmod(rows_per_sub, CH)
        for k in range(nfull):
            pltpu.sync_copy(msg_v, acc_s.at[pl.ds(row0 + k * CH, CH)])
        if rem:
            pltpu.sync_copy(msg_v.at[pl.ds(0, rem)],
                            acc_s.at[pl.ds(row0 + nfull * CH, rem)])

        plsc.subcore_barrier()

        abase = jnp.full((16,), cid * half, jnp.int32)
        not15 = iota16 != jnp.full((16,), 15, jnp.int32)

        def chunk_body(ci, carry):
            pltpu.sync_copy(feat_hbm.at[dst_v.at[ci]], msg_v)

            for j in range(CH // 16):
                base = j * 16
                s16 = src_v[ci, pl.ds(base, 16)]
                d16 = dst_v[ci, pl.ds(base, 16)]
                av = plsc.load_gather(a_v, [s16 + abase])
                cv = plsc.load_gather(c_v, [d16])
                x = av + cv
                p16 = jnp.exp(jnp.maximum(x, 0.2 * x))

                if j < (CH // 16) - 1:
                    s16n = src_v[ci, pl.ds(base + 1, 16)]
                else:
                    s16n = srcnx_v[ci, pl.ds(0, 16)]
                cum = plsc.cumsum(p16)
                m = (s16 != s16n) | (~not15)
                msub = m & not15
                plsc.addupdate_scatter(sums_v, [s16], cum, mask=m)
                plsc.addupdate_scatter(sums_v, [s16n], -cum, mask=msub)

                for l in range(16):
                    ps = jnp.full((16,), p16[l], jnp.float32)
                    for v in range(hdim // 16):
                        sl = pl.ds(v * 16, 16)
                        msg_v[base + l, sl] = msg_v[base + l, sl] * ps

            pltpu.sync_copy(msg_v, acc_s.at[src_v.at[ci]], add=True)
            return carry

        lax.fori_loop(0, nchw, chunk_body, 0)

        plsc.subcore_barrier()

        pltpu.sync_copy(acc_s.at[pl.ds(row0, rows_per_sub)],
                        acc_hbm.at[cid, sid])
        pltpu.sync_copy(sums_v, sums_hbm.at[cid, sid])

    return sc_kernel


def kernel(node_features, edges, W_hidden, b_hidden, W_att, b_att):
    n, d = node_features.shape
    h = W_hidden.shape[1]
    e = edges.shape[0]

    npad = ((n + 1 + BLK - 1) // BLK) * BLK
    half = npad // 2
    accr = ((half + 1 + NSUB - 1) // NSUB) * NSUB
    ntab = npad + 16
    slots = MAXC * CH

    xp = jnp.pad(node_features, ((0, npad - n), (0, 0)))
    wa = W_att.reshape(2, h).T
    bh2 = b_hidden.reshape(1, h)
    ba2 = jnp.concatenate([b_att, jnp.zeros((1,), jnp.float32)]).reshape(1, 2)

    src = edges[:, 0]
    dst = edges[:, 1]
    bnd = jnp.sum((src < half).astype(jnp.int32))
    cnt = jnp.stack([bnd, e - bnd])
    per_w = (cnt + NSUB - 1) // NSUB
    w_idx = jnp.arange(NSUB, dtype=jnp.int32)[None, :]
    j_idx = jnp.arange(slots, dtype=jnp.int32)[None, :]
    off = jnp.stack([jnp.zeros((), jnp.int32), bnd])[:, None]
    vw = jnp.clip(cnt[:, None] - w_idx * per_w[:, None],
                  0, per_w[:, None])
    starts = (off + w_idx * per_w[:, None]).reshape(-1)
    src_x = jnp.concatenate([src, jnp.zeros((slots,), jnp.int32)])
    dst_x = jnp.concatenate([dst, jnp.zeros((slots,), jnp.int32)])

    def _slice(st):
        return (lax.dynamic_slice(src_x, (st,), (slots,)),
                lax.dynamic_slice(dst_x, (st,), (slots,)))

    sl_src, sl_dst = lax.map(_slice, starts)
    sl_src = sl_src.reshape(NCORES, NSUB, slots)
    sl_dst = sl_dst.reshape(NCORES, NSUB, slots)
    valid = j_idx[None] < vw[:, :, None]
    srcl = jnp.where(
        valid,
        sl_src - jnp.arange(NCORES, dtype=jnp.int32)[:, None, None] * half,
        half).astype(jnp.int32)
    dsta = jnp.where(valid, sl_dst, n).astype(jnp.int32)
    srcsh = jnp.concatenate(
        [srcl[:, :, 1:], jnp.full((NCORES, NSUB, 1), half, jnp.int32)], axis=2)
    srcnx = srcsh.reshape(NCORES, NSUB, MAXC, CH)[:, :, :, CH - 16:]
    src_p = srcl.reshape(NCORES, NSUB, MAXC, CH)
    dst_p = dsta.reshape(NCORES, NSUB, MAXC, CH)
    ncw = ((vw + CH - 1) // CH).astype(jnp.int32).reshape(-1)

    feat, ac = pl.pallas_call(
        _tc_feat_body,
        grid=(npad // BLK,),
        in_specs=[
            pl.BlockSpec((BLK, d), lambda i: (i, 0)),
            pl.BlockSpec((d, h), lambda i: (0, 0)),
            pl.BlockSpec((h, 2), lambda i: (0, 0)),
            pl.BlockSpec((1, h), lambda i: (0, 0)),
            pl.BlockSpec((1, 2), lambda i: (0, 0)),
        ],
        out_specs=[
            pl.BlockSpec((BLK, h), lambda i: (i, 0)),
            pl.BlockSpec((BLK, 2), lambda i: (i, 0)),
        ],
        out_shape=[
            jax.ShapeDtypeStruct((npad, h), jnp.float32),
            jax.ShapeDtypeStruct((npad, 2), jnp.float32),
        ],
    )(xp, W_hidden, wa, bh2, ba2)

    a_tab = jnp.pad(ac[:, 0], (0, ntab - npad))
    c_tab = jnp.pad(ac[:, 1], (0, ntab - npad))

    acc, sums = _make_sc_kernel(accr, ntab, half, h)(
        feat, a_tab, c_tab, src_p, dst_p, srcnx, ncw)
    acc = acc.reshape(NCORES, accr, h)
    acc_g = jnp.concatenate([acc[0, :half], acc[1, :half]], axis=0)
    s2 = jnp.sum(sums, axis=1)
    s_g = jnp.concatenate([s2[0, :half], s2[1, :half]], axis=0)[:, None]

    out = pl.pallas_call(
        _tc_norm_body,
        grid=(npad // BLK,),
        in_specs=[
            pl.BlockSpec((BLK, h), lambda i: (i, 0)),
            pl.BlockSpec((BLK, 1), lambda i: (i, 0)),
        ],
        out_specs=pl.BlockSpec((BLK, h), lambda i: (i, 0)),
        out_shape=jax.ShapeDtypeStruct((npad, h), jnp.float32),
    )(acc_g, s_g)

    return out[:n]


# speedup vs baseline: 48.6814x; 1.4802x over previous
import functools

import jax
import jax.numpy as jnp
from jax import lax
from jax.experimental import pallas as pl
from jax.experimental.pallas import tpu as pltpu
from jax.experimental.pallas import tpu_sc as plsc

NCORES = 2
NSUB = 16
CH = 64
BLK = 512
MAXC = 157


def _tc_feat_body(x_ref, w_ref, wa_ref, bh_ref, ba_ref, feat_ref, ac_ref):
    f = jnp.dot(x_ref[...], w_ref[...], preferred_element_type=jnp.float32)
    f = f + bh_ref[...]
    feat_ref[...] = f
    ac_ref[...] = (
        jnp.dot(f, wa_ref[...], preferred_element_type=jnp.float32) + ba_ref[...]
    )


def _tc_norm_body(acc_ref, s_ref, out_ref):
    s = s_ref[...]
    out_ref[...] = jnp.where(s > 0.0, acc_ref[...] / s, 0.0)


def _make_sc_kernel(accr, ntab, half, hdim):
    mesh = plsc.VectorSubcoreMesh(
        core_axis_name="c", subcore_axis_name="s",
        num_cores=NCORES, num_subcores=NSUB,
    )
    rows_per_sub = accr // NSUB

    @functools.partial(
        pl.kernel,
        out_type=[
            jax.ShapeDtypeStruct(
                (NCORES, NSUB, rows_per_sub, hdim), jnp.float32),
            jax.ShapeDtypeStruct((NCORES, NSUB, accr), jnp.float32),
        ],
        mesh=mesh,
        compiler_params=pltpu.CompilerParams(
            needs_layout_passes=False, use_tc_tiling_on_sc=False),
        scratch_types=[
            pltpu.VMEM((ntab,), jnp.float32),
            pltpu.VMEM((ntab,), jnp.float32),
            pltpu.VMEM((32,), jnp.int32),
            pltpu.VMEM((MAXC, CH), jnp.int32),
            pltpu.VMEM((MAXC, CH), jnp.int32),
            pltpu.VMEM((MAXC, 16), jnp.int32),
            pltpu.VMEM((CH, hdim), jnp.float32),
            pltpu.VMEM((accr,), jnp.float32),
            pltpu.VMEM_SHARED((accr, hdim), jnp.float32),
        ],
    )
    def sc_kernel(feat_hbm, a_hbm, c_hbm, src_hbm, dst_hbm, srcnx_hbm,
                  ncw_hbm, acc_hbm, sums_hbm, a_v, c_v, ncw_v, src_v, dst_v,
                  srcnx_v, msg_v, sums_v, acc_s):
        cid = lax.axis_index("c")
        sid = lax.axis_index("s")

        pltpu.sync_copy(a_hbm, a_v)
        pltpu.sync_copy(c_hbm, c_v)
        pltpu.sync_copy(src_hbm.at[cid, sid], src_v)
        pltpu.sync_copy(dst_hbm.at[cid, sid], dst_v)
        pltpu.sync_copy(srcnx_hbm.at[cid, sid], srcnx_v)
        pltpu.sync_copy(ncw_hbm, ncw_v)

        iota16 = lax.iota(jnp.int32, 16)
        sid16 = jnp.full((16,), sid, jnp.int32)
        w16 = jnp.where(iota16 == sid16,
                        ncw_v[pl.ds(0, 16)] * (1 - cid) +
                        ncw_v[pl.ds(16, 16)] * cid,
                        0)
        nchw = jnp.sum(w16, axis=0)

        zero16 = jnp.zeros((16,), jnp.float32)

        def zrow(i, carry):
            for v in range(hdim // 16):
                msg_v[i, pl.ds(v * 16, 16)] = zero16
            return carry

        lax.fori_loop(0, CH, zrow, 0)

        def zsum(i, carry):
            sums_v[pl.ds(i * 16, 16)] = zero16
            return carry

        lax.fori_loop(0, accr // 16, zsum, 0)

        row0 = sid * rows_per_sub
        nfull, rem = divmod(rows_per_sub, CH)
        for k in range(nfull):
            pltpu.sync_copy(msg_v, acc_s.at[pl.ds(row0 + k * CH, CH)])
        if rem:
            pltpu.sync_copy(msg_v.at[pl.ds(0, rem)],
                            acc_s.at[pl.ds(row0 + nfull * CH, rem)])

        plsc.subcore_barrier()

        abase = jnp.full((16,), cid * half, jnp.int32)
        not15 = iota16 != jnp.full((16,), 15, jnp.int32)

        def chunk_body(ci, carry):
            pltpu.sync_copy(feat_hbm.at[dst_v.at[ci]], msg_v)

            for j in range(CH // 16):
                base = j * 16
                s16 = src_v[ci, pl.ds(base, 16)]
                d16 = dst_v[ci, pl.ds(base, 16)]
                av = plsc.load_gather(a_v, [s16 + abase])
                cv = plsc.load_gather(c_v, [d16])
                x = av + cv
                p16 = jnp.exp(jnp.maximum(x, 0.2 * x))

                if j < (CH // 16) - 1:
                    s16n = src_v[ci, pl.ds(base + 1, 16)]
                else:
                    s16n = srcnx_v[ci, pl.ds(0, 16)]
                cum = plsc.cumsum(p16)
                m = (s16 != s16n) | (~not15)
                msub = m & not15
                plsc.addupdate_scatter(sums_v, [s16], cum, mask=m)
                plsc.addupdate_scatter(sums_v, [s16n], -cum, mask=msub)

                for l in range(16):
                    ps = jnp.full((16,), p16[l], jnp.float32)
                    for v in range(hdim // 16):
                        sl = pl.ds(v * 16, 16)
                        msg_v[base + l, sl] = msg_v[base + l, sl] * ps

            pltpu.sync_copy(msg_v, acc_s.at[src_v.at[ci]], add=True)
            return carry

        lax.fori_loop(0, nchw, chunk_body, 0)

        plsc.subcore_barrier()

        pltpu.sync_copy(acc_s.at[pl.ds(row0, rows_per_sub)],
                        acc_hbm.at[cid, sid])
        pltpu.sync_copy(sums_v, sums_hbm.at[cid, sid])

    return sc_kernel


def kernel(node_features, edges, W_hidden, b_hidden, W_att, b_att):
    n, d = node_features.shape
    h = W_hidden.shape[1]
    e = edges.shape[0]

    npad = ((n + 1 + BLK - 1) // BLK) * BLK
    half = npad // 2
    accr = ((half + 1 + NSUB - 1) // NSUB) * NSUB
    ntab = npad + 16
    slots = MAXC * CH

    xp = jnp.pad(node_features, ((0, npad - n), (0, 0)))
    wa = W_att.reshape(2, h).T
    bh2 = b_hidden.reshape(1, h)
    ba2 = jnp.concatenate([b_att, jnp.zeros((1,), jnp.float32)]).reshape(1, 2)

    src = edges[:, 0]
    dst = edges[:, 1]
    bnd = jnp.sum((src < half).astype(jnp.int32))
    cnt = jnp.stack([bnd, e - bnd])
    per_w = (cnt + NSUB - 1) // NSUB
    w_idx = jnp.arange(NSUB, dtype=jnp.int32)[None, :]
    j_idx = jnp.arange(slots, dtype=jnp.int32)[None, :]
    off = jnp.stack([jnp.zeros((), jnp.int32), bnd])[:, None]
    vw = jnp.clip(cnt[:, None] - w_idx * per_w[:, None],
                  0, per_w[:, None])
    starts = (off + w_idx * per_w[:, None]).reshape(-1)
    src_x = jnp.concatenate([src, jnp.zeros((slots,), jnp.int32)])
    dst_x = jnp.concatenate([dst, jnp.zeros((slots,), jnp.int32)])

    def _slice(st):
        return (lax.dynamic_slice(src_x, (st,), (slots,)),
                lax.dynamic_slice(dst_x, (st,), (slots,)))

    parts = [_slice(starts[i]) for i in range(NCORES * NSUB)]
    sl_src = jnp.stack([p[0] for p in parts]).reshape(NCORES, NSUB, slots)
    sl_dst = jnp.stack([p[1] for p in parts]).reshape(NCORES, NSUB, slots)
    valid = j_idx[None] < vw[:, :, None]
    srcl = jnp.where(
        valid,
        sl_src - jnp.arange(NCORES, dtype=jnp.int32)[:, None, None] * half,
        half).astype(jnp.int32)
    dsta = jnp.where(valid, sl_dst, n).astype(jnp.int32)
    srcsh = jnp.concatenate(
        [srcl[:, :, 1:], jnp.full((NCORES, NSUB, 1), half, jnp.int32)], axis=2)
    srcnx = srcsh.reshape(NCORES, NSUB, MAXC, CH)[:, :, :, CH - 16:]
    src_p = srcl.reshape(NCORES, NSUB, MAXC, CH)
    dst_p = dsta.reshape(NCORES, NSUB, MAXC, CH)
    ncw = ((vw + CH - 1) // CH).astype(jnp.int32).reshape(-1)

    feat, ac = pl.pallas_call(
        _tc_feat_body,
        grid=(npad // BLK,),
        in_specs=[
            pl.BlockSpec((BLK, d), lambda i: (i, 0)),
            pl.BlockSpec((d, h), lambda i: (0, 0)),
            pl.BlockSpec((h, 2), lambda i: (0, 0)),
            pl.BlockSpec((1, h), lambda i: (0, 0)),
            pl.BlockSpec((1, 2), lambda i: (0, 0)),
        ],
        out_specs=[
            pl.BlockSpec((BLK, h), lambda i: (i, 0)),
            pl.BlockSpec((BLK, 2), lambda i: (i, 0)),
        ],
        out_shape=[
            jax.ShapeDtypeStruct((npad, h), jnp.float32),
            jax.ShapeDtypeStruct((npad, 2), jnp.float32),
        ],
    )(xp, W_hidden, wa, bh2, ba2)

    a_tab = jnp.pad(ac[:, 0], (0, ntab - npad))
    c_tab = jnp.pad(ac[:, 1], (0, ntab - npad))

    acc, sums = _make_sc_kernel(accr, ntab, half, h)(
        feat, a_tab, c_tab, src_p, dst_p, srcnx, ncw)
    acc = acc.reshape(NCORES, accr, h)
    acc_g = jnp.concatenate([acc[0, :half], acc[1, :half]], axis=0)
    s2 = jnp.sum(sums, axis=1)
    s_g = jnp.concatenate([s2[0, :half], s2[1, :half]], axis=0)[:, None]

    out = pl.pallas_call(
        _tc_norm_body,
        grid=(npad // BLK,),
        in_specs=[
            pl.BlockSpec((BLK, h), lambda i: (i, 0)),
            pl.BlockSpec((BLK, 1), lambda i: (i, 0)),
        ],
        out_specs=pl.BlockSpec((BLK, h), lambda i: (i, 0)),
        out_shape=jax.ShapeDtypeStruct((npad, h), jnp.float32),
    )(acc_g, s_g)

    return out[:n]


# speedup vs baseline: 71.7342x; 1.4735x over previous
import functools

import jax
import jax.numpy as jnp
from jax import lax
from jax.experimental import pallas as pl
from jax.experimental.pallas import tpu as pltpu
from jax.experimental.pallas import tpu_sc as plsc

NCORES = 2
NSUB = 16
CH = 64
BLK = 512
MAXC = 157


def _tc_feat_body(x_ref, w_ref, wa_ref, bh_ref, ba_ref, feat_ref, ac_ref):
    f = jnp.dot(x_ref[...], w_ref[...], preferred_element_type=jnp.float32)
    f = f + bh_ref[...]
    feat_ref[...] = f
    ac_ref[...] = (
        jnp.dot(f, wa_ref[...], preferred_element_type=jnp.float32) + ba_ref[...]
    )


def _tc_norm_body(acc_ref, s_ref, out_ref):
    s = s_ref[...]
    out_ref[...] = jnp.where(s > 0.0, acc_ref[...] / s, 0.0)


def _make_sc_kernel(accr, ntab, half, hdim):
    mesh = plsc.VectorSubcoreMesh(
        core_axis_name="c", subcore_axis_name="s",
        num_cores=NCORES, num_subcores=NSUB,
    )
    rows_per_sub = accr // NSUB

    @functools.partial(
        pl.kernel,
        out_type=[
            jax.ShapeDtypeStruct(
                (NCORES, NSUB, rows_per_sub, hdim), jnp.float32),
            jax.ShapeDtypeStruct((NCORES, NSUB, accr), jnp.float32),
        ],
        mesh=mesh,
        compiler_params=pltpu.CompilerParams(
            needs_layout_passes=False, use_tc_tiling_on_sc=False),
        scratch_types=[
            pltpu.VMEM((ntab,), jnp.float32),
            pltpu.VMEM((ntab,), jnp.float32),
            pltpu.VMEM((32,), jnp.int32),
            pltpu.VMEM((MAXC, CH), jnp.int32),
            pltpu.VMEM((MAXC, CH), jnp.int32),
            pltpu.VMEM((MAXC, 16), jnp.int32),
            pltpu.VMEM((CH, hdim), jnp.float32),
            pltpu.VMEM((accr,), jnp.float32),
            pltpu.VMEM_SHARED((accr, hdim), jnp.float32),
        ],
    )
    def sc_kernel(feat_hbm, a_hbm, c_hbm, src_hbm, dst_hbm, srcnx_hbm,
                  ncw_hbm, acc_hbm, sums_hbm, a_v, c_v, ncw_v, src_v, dst_v,
                  srcnx_v, msg_v, sums_v, acc_s):
        cid = lax.axis_index("c")
        sid = lax.axis_index("s")

        pltpu.sync_copy(a_hbm, a_v)
        pltpu.sync_copy(c_hbm, c_v)
        pltpu.sync_copy(src_hbm.at[cid, sid], src_v)
        pltpu.sync_copy(dst_hbm.at[cid, sid], dst_v)
        pltpu.sync_copy(srcnx_hbm.at[cid, sid], srcnx_v)
        pltpu.sync_copy(ncw_hbm, ncw_v)

        iota16 = lax.iota(jnp.int32, 16)
        sid16 = jnp.full((16,), sid, jnp.int32)
        w16 = jnp.where(iota16 == sid16,
                        ncw_v[pl.ds(0, 16)] * (1 - cid) +
                        ncw_v[pl.ds(16, 16)] * cid,
                        0)
        nchw = jnp.sum(w16, axis=0)

        zero16 = jnp.zeros((16,), jnp.float32)

        def zrow(i, carry):
            for v in range(hdim // 16):
                msg_v[i, pl.ds(v * 16, 16)] = zero16
            return carry

        lax.fori_loop(0, CH, zrow, 0)

        def zsum(i, carry):
            sums_v[pl.ds(i * 16, 16)] = zero16
            return carry

        lax.fori_loop(0, accr // 16, zsum, 0)

        row0 = sid * rows_per_sub
        nfull, rem = divmod(rows_per_sub, CH)
        for k in range(nfull):
            pltpu.sync_copy(msg_v, acc_s.at[pl.ds(row0 + k * CH, CH)])
        if rem:
            pltpu.sync_copy(msg_v.at[pl.ds(0, rem)],
                            acc_s.at[pl.ds(row0 + nfull * CH, rem)])

        plsc.subcore_barrier()

        abase = jnp.full((16,), cid * half, jnp.int32)
        not15 = iota16 != jnp.full((16,), 15, jnp.int32)

        def chunk_body(ci, carry):
            pltpu.sync_copy(feat_hbm.at[dst_v.at[ci]], msg_v)

            for j in range(CH // 16):
                base = j * 16
                s16 = src_v[ci, pl.ds(base, 16)]
                d16 = dst_v[ci, pl.ds(base, 16)]
                av = plsc.load_gather(a_v, [s16 + abase])
                cv = plsc.load_gather(c_v, [d16])
                x = av + cv
                p16 = jnp.exp(jnp.maximum(x, 0.2 * x))

                if j < (CH // 16) - 1:
                    s16n = src_v[ci, pl.ds(base + 1, 16)]
                else:
                    s16n = srcnx_v[ci, pl.ds(0, 16)]
                cum = plsc.cumsum(p16)
                m = (s16 != s16n) | (~not15)
                msub = m & not15
                plsc.addupdate_scatter(sums_v, [s16], cum, mask=m)
                plsc.addupdate_scatter(sums_v, [s16n], -cum, mask=msub)

                for l in range(16):
                    ps = jnp.full((16,), p16[l], jnp.float32)
                    for v in range(hdim // 16):
                        sl = pl.ds(v * 16, 16)
                        msg_v[base + l, sl] = msg_v[base + l, sl] * ps

            pltpu.sync_copy(msg_v, acc_s.at[src_v.at[ci]], add=True)
            return carry

        lax.fori_loop(0, nchw, chunk_body, 0)

        plsc.subcore_barrier()

        pltpu.sync_copy(acc_s.at[pl.ds(row0, rows_per_sub)],
                        acc_hbm.at[cid, sid])
        pltpu.sync_copy(sums_v, sums_hbm.at[cid, sid])

    return sc_kernel


def kernel(node_features, edges, W_hidden, b_hidden, W_att, b_att):
    n, d = node_features.shape
    h = W_hidden.shape[1]
    e = edges.shape[0]

    npad = ((n + 1 + BLK - 1) // BLK) * BLK
    half = npad // 2
    accr = ((half + 1 + NSUB - 1) // NSUB) * NSUB
    ntab = npad + 16
    slots = MAXC * CH

    xp = jnp.pad(node_features, ((0, npad - n), (0, 0)))
    wa = W_att.reshape(2, h).T
    bh2 = b_hidden.reshape(1, h)
    ba2 = jnp.concatenate([b_att, jnp.zeros((1,), jnp.float32)]).reshape(1, 2)

    src = edges[:, 0]
    dst = edges[:, 1]
    bnd = jnp.sum((src < half).astype(jnp.int32))
    cnt = jnp.stack([bnd, e - bnd])
    off = jnp.stack([jnp.zeros((), jnp.int32), bnd])
    kc = (cnt + CH - 1) // CH
    w_idx = jnp.arange(NSUB, dtype=jnp.int32)[None, :]
    ncw = jnp.maximum((kc[:, None] - w_idx + NSUB - 1) // NSUB, 0)
    slots16 = NSUB * slots
    pad1 = jnp.zeros((slots16 + 8,), jnp.int32)
    src_x = jnp.concatenate([src, pad1])
    dst_x = jnp.concatenate([dst, pad1])

    def _grab(x_pad, c):
        sl = lax.dynamic_slice(x_pad, (off[c],), (slots16,))
        sh = jnp.concatenate([sl[1:], jnp.zeros((1,), jnp.int32)])
        return sl, sh

    lanes = jnp.arange(CH, dtype=jnp.int32)[None, None, :]
    chk = jnp.arange(MAXC, dtype=jnp.int32)[None, :, None]
    gpos = (chk * NSUB + jnp.arange(NSUB, dtype=jnp.int32)[:, None, None]) * CH

    srcl_l, dsta_l, srcnx_l = [], [], []
    for c in range(NCORES):
        ssl, ssh = _grab(src_x, c)
        dsl, _ = _grab(dst_x, c)
        s3 = ssl.reshape(MAXC * NSUB, CH).reshape(MAXC, NSUB, CH).transpose(1, 0, 2)
        sh3 = ssh.reshape(MAXC, NSUB, CH).transpose(1, 0, 2)
        d3 = dsl.reshape(MAXC, NSUB, CH).transpose(1, 0, 2)
        valid = (gpos + lanes) < cnt[c]
        validn = (gpos + lanes + 1) < cnt[c]
        srcl_l.append(jnp.where(valid, s3 - c * half, half))
        dsta_l.append(jnp.where(valid, d3, n))
        srcnx_l.append(jnp.where(validn, sh3, half + c * half)[:, :, CH - 16:]
                       - c * half)
    src_p = jnp.stack(srcl_l).astype(jnp.int32)
    dst_p = jnp.stack(dsta_l).astype(jnp.int32)
    srcnx = jnp.stack(srcnx_l).astype(jnp.int32)
    ncw = ncw.astype(jnp.int32).reshape(-1)

    feat, ac = pl.pallas_call(
        _tc_feat_body,
        grid=(npad // BLK,),
        in_specs=[
            pl.BlockSpec((BLK, d), lambda i: (i, 0)),
            pl.BlockSpec((d, h), lambda i: (0, 0)),
            pl.BlockSpec((h, 2), lambda i: (0, 0)),
            pl.BlockSpec((1, h), lambda i: (0, 0)),
            pl.BlockSpec((1, 2), lambda i: (0, 0)),
        ],
        out_specs=[
            pl.BlockSpec((BLK, h), lambda i: (i, 0)),
            pl.BlockSpec((BLK, 2), lambda i: (i, 0)),
        ],
        out_shape=[
            jax.ShapeDtypeStruct((npad, h), jnp.float32),
            jax.ShapeDtypeStruct((npad, 2), jnp.float32),
        ],
    )(xp, W_hidden, wa, bh2, ba2)

    a_tab = jnp.pad(ac[:, 0], (0, ntab - npad))
    c_tab = jnp.pad(ac[:, 1], (0, ntab - npad))

    acc, sums = _make_sc_kernel(accr, ntab, half, h)(
        feat, a_tab, c_tab, src_p, dst_p, srcnx, ncw)
    acc = acc.reshape(NCORES, accr, h)
    acc_g = jnp.concatenate([acc[0, :half], acc[1, :half]], axis=0)
    s2 = jnp.sum(sums, axis=1)
    s_g = jnp.concatenate([s2[0, :half], s2[1, :half]], axis=0)[:, None]

    out = pl.pallas_call(
        _tc_norm_body,
        grid=(npad // BLK,),
        in_specs=[
            pl.BlockSpec((BLK, h), lambda i: (i, 0)),
            pl.BlockSpec((BLK, 1), lambda i: (i, 0)),
        ],
        out_specs=pl.BlockSpec((BLK, h), lambda i: (i, 0)),
        out_shape=jax.ShapeDtypeStruct((npad, h), jnp.float32),
    )(acc_g, s_g)

    return out[:n]
